# baseline (speedup 1.0000x reference)
import functools
import jax
import jax.numpy as jnp
from jax.experimental import pallas as pl
from jax.experimental.pallas import tpu as pltpu

N = 10000
E = 320000
D = 128
ALPHA = 0.2

N_BLK = 1000


def _proj_body(x_ref, wt_ref, a_ref, h_ref, s_ref):
    h = jnp.dot(x_ref[...], wt_ref[...], preferred_element_type=jnp.float32)
    h_ref[...] = h
    s_ref[...] = jnp.dot(h, a_ref[...], preferred_element_type=jnp.float32)


def _project(x, wt, a2):
    grid = (N // N_BLK,)
    return pl.pallas_call(
        _proj_body,
        grid=grid,
        in_specs=[
            pl.BlockSpec((N_BLK, D), lambda i: (i, 0)),
            pl.BlockSpec((D, D), lambda i: (0, 0)),
            pl.BlockSpec((D, 2), lambda i: (0, 0)),
        ],
        out_specs=[
            pl.BlockSpec((N_BLK, D), lambda i: (i, 0)),
            pl.BlockSpec((N_BLK, 2), lambda i: (i, 0)),
        ],
        out_shape=[
            jax.ShapeDtypeStruct((N, D), jnp.float32),
            jax.ShapeDtypeStruct((N, 2), jnp.float32),
        ],
    )(x, wt, a2)


def _epi_body(hp_ref, den_ref, out_ref):
    hsum = hp_ref[0] + hp_ref[1]
    den = den_ref[0] + den_ref[1] + 1e-10
    h = hsum / den
    out_ref[...] = jnp.where(h > 0.0, h, jnp.exp(h) - 1.0)


def _epilogue(hp, den, n_rows, blk):
    grid = (n_rows // blk,)
    return pl.pallas_call(
        _epi_body,
        grid=grid,
        in_specs=[
            pl.BlockSpec((2, blk, D), lambda i: (0, i, 0)),
            pl.BlockSpec((2, blk, 1), lambda i: (0, i, 0)),
        ],
        out_specs=pl.BlockSpec((blk, D), lambda i: (i, 0)),
        out_shape=jax.ShapeDtypeStruct((n_rows, D), jnp.float32),
    )(hp, den)


def kernel(node_features, edge_index, W, a):
    x = node_features
    wt = W.T
    a2 = jnp.reshape(a, (2, D)).T

    h, s = _project(x, wt, a2)
    s1 = s[:, 0]
    s2 = s[:, 1]

    src = edge_index[0]
    tgt = edge_index[1]

    e = s1[src] + s2[tgt]
    e = jnp.where(e > 0, e, ALPHA * e)
    p = jnp.exp(e - jnp.max(e))
    denom = jax.ops.segment_sum(p, tgt, num_segments=N)
    h_raw = jax.ops.segment_sum(p[:, None] * jnp.take(h, src, axis=0), tgt,
                                num_segments=N)
    hp = jnp.stack([h_raw, jnp.zeros_like(h_raw)])
    den = jnp.stack([denom, jnp.zeros_like(denom)])[:, :, None]

    return _epilogue(hp, den, N, N_BLK)


# speedup vs baseline: 8.1085x; 8.1085x over previous
import functools
import jax
import jax.numpy as jnp
from jax import lax
from jax.experimental import pallas as pl
from jax.experimental.pallas import tpu as pltpu
from jax.experimental.pallas import tpu_sc as plsc

N = 10000
E = 320000
D = 128
ALPHA = 0.2

NC = 2
NS = 16
NW = NC * NS

EPW = E // NW
GW = 128
GROUPS = 79
EPW_PAD = GROUPS * GW
NROW_PAD = 10240
RPT = NROW_PAD // NS

N_BLK = 1000

_NEG = -1e30


def _proj_body(x_ref, wt_ref, a_ref, h_ref, s_ref):
    h = jnp.dot(x_ref[...], wt_ref[...], preferred_element_type=jnp.float32)
    h_ref[...] = h
    s_ref[...] = jnp.dot(h, a_ref[...], preferred_element_type=jnp.float32)


def _project(x, wt, a2):
    grid = (N // N_BLK,)
    return pl.pallas_call(
        _proj_body,
        grid=grid,
        in_specs=[
            pl.BlockSpec((N_BLK, D), lambda i: (i, 0)),
            pl.BlockSpec((D, D), lambda i: (0, 0)),
            pl.BlockSpec((D, 2), lambda i: (0, 0)),
        ],
        out_specs=[
            pl.BlockSpec((N_BLK, D), lambda i: (i, 0)),
            pl.BlockSpec((N_BLK, 2), lambda i: (i, 0)),
        ],
        out_shape=[
            jax.ShapeDtypeStruct((N, D), jnp.float32),
            jax.ShapeDtypeStruct((N, 2), jnp.float32),
        ],
    )(x, wt, a2)


def _logits_body(s1_hbm, s2_hbm, src_hbm, tgt_hbm, e_hbm, mx_hbm,
                 s1_v, s2_v, src_v, tgt_v, e_v, mx_v):
    c = lax.axis_index("c")
    s = lax.axis_index("s")
    wid = c * NS + s

    pltpu.sync_copy(s1_hbm, s1_v)
    pltpu.sync_copy(s2_hbm, s2_v)
    pltpu.sync_copy(src_hbm.at[wid], src_v)
    pltpu.sync_copy(tgt_hbm.at[wid], tgt_v)

    def one(off, macc):
        sv = src_v[pl.ds(off, 16)]
        tv = tgt_v[pl.ds(off, 16)]
        v = plsc.load_gather(s1_v, [sv]) + plsc.load_gather(s2_v, [tv])
        e = jnp.maximum(v, ALPHA * v)
        e_v[pl.ds(off, 16)] = e
        return jnp.maximum(macc, e)

    def body(j, macc):
        base = j * 128
        for k in range(8):
            macc = one(base + 16 * k, macc)
        return macc

    macc = jnp.full((16,), _NEG, jnp.float32)
    macc = lax.fori_loop(0, EPW // 128, body, macc)
    macc = one(EPW - 16, macc)
    pad = jnp.full((16,), _NEG, jnp.float32)
    for k in range((EPW_PAD - EPW) // 16):
        e_v[pl.ds(EPW + 16 * k, 16)] = pad

    mx_v[...] = macc
    pltpu.sync_copy(e_v, e_hbm.at[wid])
    pltpu.sync_copy(mx_v, mx_hbm.at[pl.ds(wid * 16, 16)])


def _logits(s1, s2, srcp, tgtp):
    mesh = plsc.VectorSubcoreMesh(core_axis_name="c", subcore_axis_name="s",
                                  num_cores=NC, num_subcores=NS)
    f = pl.kernel(
        _logits_body,
        out_type=[
            jax.ShapeDtypeStruct((NW, EPW_PAD), jnp.float32),
            jax.ShapeDtypeStruct((NW * 16,), jnp.float32),
        ],
        mesh=mesh,
        scratch_types=[
            pltpu.VMEM((N,), jnp.float32),
            pltpu.VMEM((N,), jnp.float32),
            pltpu.VMEM((EPW_PAD,), jnp.int32),
            pltpu.VMEM((EPW_PAD,), jnp.int32),
            pltpu.VMEM((EPW_PAD,), jnp.float32),
            pltpu.VMEM((16,), jnp.float32),
        ],
        compiler_params=pltpu.CompilerParams(needs_layout_passes=False),
    )
    return f(s1, s2, srcp, tgtp)


def _agg_body(h_hbm, e_hbm, src_hbm, tgt_hbm, mx_hbm, zrow_hbm,
              hp_hbm, den_hbm,
              src2d, tgt2d, ep_v, mx_v, rows_v, dz_v, hp_sh, den_sh, sem):
    c = lax.axis_index("c")
    s = lax.axis_index("s")
    wid = c * NS + s

    pltpu.sync_copy(src_hbm.at[wid], src2d)
    pltpu.sync_copy(tgt_hbm.at[wid], tgt2d)
    pltpu.sync_copy(e_hbm.at[wid], ep_v)
    pltpu.sync_copy(mx_hbm, mx_v)

    def mx_body(i, macc):
        return jnp.maximum(macc, mx_v[pl.ds(i * 16, 16)])

    macc = lax.fori_loop(0, NW, mx_body, jnp.full((16,), _NEG, jnp.float32))
    m = jnp.max(macc)

    def p_body(j, _):
        ep_v[pl.ds(j * 16, 16)] = jnp.exp(ep_v[pl.ds(j * 16, 16)] - m)
        return 0

    lax.fori_loop(0, EPW_PAD // 16, p_body, 0)

    rslc = pl.ds(s * RPT, RPT)
    pltpu.sync_copy(zrow_hbm.at[rslc], hp_sh.at[rslc])
    zv = jnp.zeros((16,), jnp.float32)

    def z_body(j, _):
        dz_v[pl.ds(j * 16, 16)] = zv
        return 0

    lax.fori_loop(0, RPT // 16, z_body, 0)
    pltpu.sync_copy(dz_v, den_sh.at[rslc])
    plsc.subcore_barrier()

    iota = lax.iota(jnp.int32, 16)
    cols = [iota + 16 * k for k in range(8)]

    def grp(g, _):
        pltpu.async_copy(h_hbm.at[src2d.at[g]], rows_v, sem).wait()
        pltpu.sync_copy(ep_v.at[pl.ds(g * GW, GW)],
                        den_sh.at[tgt2d.at[g]], add=True)

        def row(r, _):
            pv = plsc.load_gather(ep_v, [jnp.full((16,), g * GW + r,
                                                  jnp.int32)])
            rv = jnp.full((16,), r, jnp.int32)
            for k in range(8):
                x = plsc.load_gather(rows_v, [rv, cols[k]])
                plsc.store_scatter(rows_v, [rv, cols[k]], x * pv)
            return 0

        lax.fori_loop(0, GW, row, 0)
        pltpu.sync_copy(rows_v, hp_sh.at[tgt2d.at[g]], add=True)
        return 0

    lax.fori_loop(0, GROUPS, grp, 0)
    plsc.subcore_barrier()

    pltpu.sync_copy(hp_sh.at[rslc], hp_hbm.at[c, rslc])
    pltpu.sync_copy(den_sh.at[rslc], den_hbm.at[c, rslc])


def _aggregate(h, e, srcp2, tgtp2, mx, zrow):
    mesh = plsc.VectorSubcoreMesh(core_axis_name="c", subcore_axis_name="s",
                                  num_cores=NC, num_subcores=NS)
    f = pl.kernel(
        _agg_body,
        out_type=[
            jax.ShapeDtypeStruct((NC, NROW_PAD, D), jnp.float32),
            jax.ShapeDtypeStruct((NC, NROW_PAD), jnp.float32),
        ],
        mesh=mesh,
        scratch_types=[
            pltpu.VMEM((GROUPS, GW), jnp.int32),
            pltpu.VMEM((GROUPS, GW), jnp.int32),
            pltpu.VMEM((EPW_PAD,), jnp.float32),
            pltpu.VMEM((NW * 16,), jnp.float32),
            pltpu.VMEM((GW, D), jnp.float32),
            pltpu.VMEM((RPT,), jnp.float32),
            pltpu.VMEM_SHARED((NROW_PAD, D), jnp.float32),
            pltpu.VMEM_SHARED((NROW_PAD,), jnp.float32),
            pltpu.SemaphoreType.DMA,
        ],
        compiler_params=pltpu.CompilerParams(needs_layout_passes=False),
    )
    return f(h, e, srcp2, tgtp2, mx, zrow)


def _epi_body(hp_ref, den_ref, out_ref):
    hsum = hp_ref[0] + hp_ref[1]
    den = den_ref[0] + den_ref[1] + 1e-10
    h = hsum / den
    out_ref[...] = jnp.where(h > 0.0, h, jnp.exp(h) - 1.0)


def _epilogue(hp, den, n_rows, blk):
    grid = (n_rows // blk,)
    return pl.pallas_call(
        _epi_body,
        grid=grid,
        in_specs=[
            pl.BlockSpec((2, blk, D), lambda i: (0, i, 0)),
            pl.BlockSpec((2, blk, 1), lambda i: (0, i, 0)),
        ],
        out_specs=pl.BlockSpec((blk, D), lambda i: (i, 0)),
        out_shape=jax.ShapeDtypeStruct((n_rows, D), jnp.float32),
    )(hp, den)


def kernel(node_features, edge_index, W, a):
    x = node_features
    wt = W.T
    a2 = jnp.reshape(a, (2, D)).T

    h, sca = _project(x, wt, a2)
    s1 = sca[:, 0]
    s2 = sca[:, 1]

    src = jnp.reshape(edge_index[0], (NW, EPW))
    tgt = jnp.reshape(edge_index[1], (NW, EPW))
    srcp = jnp.pad(src, ((0, 0), (0, EPW_PAD - EPW)))
    tgtp = jnp.pad(tgt, ((0, 0), (0, EPW_PAD - EPW)), constant_values=N)

    e, mx = _logits(s1, s2, srcp, tgtp)

    srcp2 = jnp.reshape(srcp, (NW, GROUPS, GW))
    tgtp2 = jnp.reshape(tgtp, (NW, GROUPS, GW))
    zrow = jnp.zeros((NROW_PAD, D), jnp.float32)
    hp, den = _aggregate(h, e, srcp2, tgtp2, mx, zrow)

    out = _epilogue(hp, den[:, :, None], NROW_PAD, 1024)
    return out[:N]


# speedup vs baseline: 17.1382x; 2.1136x over previous
import functools
import jax
import jax.numpy as jnp
from jax import lax
from jax.experimental import pallas as pl
from jax.experimental.pallas import tpu as pltpu
from jax.experimental.pallas import tpu_sc as plsc

N = 10000
E = 320000
D = 128
ALPHA = 0.2

NC = 2
NS = 16
NW = NC * NS

EPW = E // NW
GW = 128
GROUPS = 79
EPW_PAD = GROUPS * GW
NROW_PAD = 10240
RPT = NROW_PAD // NS

N_BLK = 1000

_NEG = -1e30


def _proj_body(x_ref, wt_ref, a_ref, h_ref, s_ref):
    h = jnp.dot(x_ref[...], wt_ref[...], preferred_element_type=jnp.float32)
    h_ref[...] = h
    s_ref[...] = jnp.dot(h, a_ref[...], preferred_element_type=jnp.float32)


def _project(x, wt, a2):
    grid = (N // N_BLK,)
    return pl.pallas_call(
        _proj_body,
        grid=grid,
        in_specs=[
            pl.BlockSpec((N_BLK, D), lambda i: (i, 0)),
            pl.BlockSpec((D, D), lambda i: (0, 0)),
            pl.BlockSpec((D, 2), lambda i: (0, 0)),
        ],
        out_specs=[
            pl.BlockSpec((N_BLK, D), lambda i: (i, 0)),
            pl.BlockSpec((N_BLK, 2), lambda i: (i, 0)),
        ],
        out_shape=[
            jax.ShapeDtypeStruct((N, D), jnp.float32),
            jax.ShapeDtypeStruct((N, 2), jnp.float32),
        ],
    )(x, wt, a2)


def _logits_body(s1_hbm, s2_hbm, src_hbm, tgt_hbm, e_hbm, mx_hbm,
                 s1_v, s2_v, src_v, tgt_v, e_v, mx_v):
    c = lax.axis_index("c")
    s = lax.axis_index("s")
    wid = c * NS + s

    pltpu.sync_copy(s1_hbm, s1_v)
    pltpu.sync_copy(s2_hbm, s2_v)
    pltpu.sync_copy(src_hbm.at[wid], src_v)
    pltpu.sync_copy(tgt_hbm.at[wid], tgt_v)

    def one(off, macc):
        sv = src_v[pl.ds(off, 16)]
        tv = tgt_v[pl.ds(off, 16)]
        v = plsc.load_gather(s1_v, [sv]) + plsc.load_gather(s2_v, [tv])
        e = jnp.maximum(v, ALPHA * v)
        e_v[pl.ds(off, 16)] = e
        return jnp.maximum(macc, e)

    def body(j, macc):
        base = j * 128
        for k in range(8):
            macc = one(base + 16 * k, macc)
        return macc

    macc = jnp.full((16,), _NEG, jnp.float32)
    macc = lax.fori_loop(0, EPW // 128, body, macc)
    macc = one(EPW - 16, macc)
    pad = jnp.full((16,), _NEG, jnp.float32)
    for k in range((EPW_PAD - EPW) // 16):
        e_v[pl.ds(EPW + 16 * k, 16)] = pad

    mx_v[...] = macc
    pltpu.sync_copy(e_v, e_hbm.at[wid])
    pltpu.sync_copy(mx_v, mx_hbm.at[pl.ds(wid * 16, 16)])


def _logits(s1, s2, srcp, tgtp):
    mesh = plsc.VectorSubcoreMesh(core_axis_name="c", subcore_axis_name="s",
                                  num_cores=NC, num_subcores=NS)
    f = pl.kernel(
        _logits_body,
        out_type=[
            jax.ShapeDtypeStruct((NW, EPW_PAD), jnp.float32),
            jax.ShapeDtypeStruct((NW * 16,), jnp.float32),
        ],
        mesh=mesh,
        scratch_types=[
            pltpu.VMEM((N,), jnp.float32),
            pltpu.VMEM((N,), jnp.float32),
            pltpu.VMEM((EPW_PAD,), jnp.int32),
            pltpu.VMEM((EPW_PAD,), jnp.int32),
            pltpu.VMEM((EPW_PAD,), jnp.float32),
            pltpu.VMEM((16,), jnp.float32),
        ],
        compiler_params=pltpu.CompilerParams(needs_layout_passes=False),
    )
    return f(s1, s2, srcp, tgtp)


def _agg_body(h_hbm, e_hbm, src_hbm, tgt_hbm, mx_hbm, zrow_hbm, zden_hbm,
              hp_hbm, den_hbm,
              mx_v, rows_a, rows_b, srcg_a, srcg_b, tgtg_a, tgtg_b,
              eg_a, eg_b, pg_a, pg_b, hp_sh, den_sh,
              gsa, gsb, psa, psb, dsem):
    c = lax.axis_index("c")
    s = lax.axis_index("s")
    wid = c * NS + s

    pltpu.sync_copy(mx_hbm, mx_v)

    def mx_body(i, macc):
        return jnp.maximum(macc, mx_v[pl.ds(i * 16, 16)])

    macc = lax.fori_loop(0, NW, mx_body, jnp.full((16,), _NEG, jnp.float32))
    m = jnp.max(macc)

    rslc = pl.ds(s * RPT, RPT)
    pltpu.sync_copy(zrow_hbm.at[rslc], hp_sh.at[rslc])
    pltpu.sync_copy(zden_hbm.at[rslc], den_sh.at[rslc])
    plsc.subcore_barrier()

    bufs = ((rows_a, srcg_a, tgtg_a, eg_a, pg_a, gsa, psa),
            (rows_b, srcg_b, tgtg_b, eg_b, pg_b, gsb, psb))

    def pf_issue(g, bb, sem):
        pltpu.async_copy(src_hbm.at[wid, g], bb[1], sem)
        pltpu.async_copy(tgt_hbm.at[wid, g], bb[2], sem)
        pltpu.async_copy(e_hbm.at[wid, pl.ds(g * GW, GW)], bb[3], sem)

    def pf_wait(g, bb, sem):
        pltpu.make_async_copy(src_hbm.at[wid, g], bb[1], sem).wait()
        pltpu.make_async_copy(tgt_hbm.at[wid, g], bb[2], sem).wait()
        pltpu.make_async_copy(e_hbm.at[wid, pl.ds(g * GW, GW)], bb[3],
                              sem).wait()

    def p_transform(bb):
        for q in range(GW // 16):
            sl = pl.ds(16 * q, 16)
            bb[4][sl] = jnp.exp(bb[3][sl] - m)

    def g_issue(g, bb):
        pltpu.async_copy(h_hbm.at[bb[1]], bb[0], bb[5])

    def g_wait(g, bb):
        pltpu.make_async_copy(h_hbm.at[bb[1]], bb[0], bb[5]).wait()

    def d_issue(bb):
        pltpu.async_copy(bb[4], den_sh.at[bb[2]], dsem, add=True)

    def d_wait(bb):
        pltpu.make_async_copy(bb[4], den_sh.at[bb[2]], dsem).wait()

    def scale(bb):
        rows, pg = bb[0], bb[4]

        def row16(mm, _):
            base = mm * 16
            pvec = pg[pl.ds(base, 16)]
            for j in range(16):
                r = base + j
                sc = jnp.full((16,), pvec[j])
                for k in range(8):
                    sl = pl.ds(16 * k, 16)
                    rows[r, sl] = rows[r, sl] * sc
            return 0

        lax.fori_loop(0, GW // 16, row16, 0)

    def s_sync(bb):
        pltpu.sync_copy(bb[0], hp_sh.at[bb[2]], add=True)

    pf_issue(0, bufs[0], bufs[0][6])
    pf_wait(0, bufs[0], bufs[0][6])
    p_transform(bufs[0])
    g_issue(0, bufs[0])
    pf_issue(1, bufs[1], bufs[1][6])

    def phase(g, bx, by):
        g_wait(g, bx)

        @pl.when(g + 1 < GROUPS)
        def _():
            pf_wait(g + 1, by, by[6])
            p_transform(by)
            g_issue(g + 1, by)

        d_issue(bx)
        scale(bx)
        s_sync(bx)

        @pl.when(g + 2 < GROUPS)
        def _():
            d_wait(bx)
            pf_issue(g + 2, bx, bx[6])

    def pair(gg, _):
        g0 = 2 * gg
        phase(g0, bufs[0], bufs[1])

        @pl.when(g0 + 1 < GROUPS)
        def _():
            phase(g0 + 1, bufs[1], bufs[0])

        return 0

    lax.fori_loop(0, (GROUPS + 1) // 2, pair, 0)
    d_wait(bufs[0])
    d_wait(bufs[1])
    plsc.subcore_barrier()

    pltpu.sync_copy(hp_sh.at[rslc], hp_hbm.at[c, rslc])
    pltpu.sync_copy(den_sh.at[rslc], den_hbm.at[c, rslc])


def _aggregate(h, e, srcp2, tgtp2, mx, zrow, zden):
    mesh = plsc.VectorSubcoreMesh(core_axis_name="c", subcore_axis_name="s",
                                  num_cores=NC, num_subcores=NS)
    f = pl.kernel(
        _agg_body,
        out_type=[
            jax.ShapeDtypeStruct((NC, NROW_PAD, D), jnp.float32),
            jax.ShapeDtypeStruct((NC, NROW_PAD), jnp.float32),
        ],
        mesh=mesh,
        scratch_types=[
            pltpu.VMEM((NW * 16,), jnp.float32),
            pltpu.VMEM((GW, D), jnp.float32),
            pltpu.VMEM((GW, D), jnp.float32),
            pltpu.VMEM((GW,), jnp.int32),
            pltpu.VMEM((GW,), jnp.int32),
            pltpu.VMEM((GW,), jnp.int32),
            pltpu.VMEM((GW,), jnp.int32),
            pltpu.VMEM((GW,), jnp.float32),
            pltpu.VMEM((GW,), jnp.float32),
            pltpu.VMEM((GW,), jnp.float32),
            pltpu.VMEM((GW,), jnp.float32),
            pltpu.VMEM_SHARED((NROW_PAD, D), jnp.float32),
            pltpu.VMEM_SHARED((NROW_PAD,), jnp.float32),
            pltpu.SemaphoreType.DMA,
            pltpu.SemaphoreType.DMA,
            pltpu.SemaphoreType.DMA,
            pltpu.SemaphoreType.DMA,
            pltpu.SemaphoreType.DMA,
        ],
        compiler_params=pltpu.CompilerParams(needs_layout_passes=False),
    )
    return f(h, e, srcp2, tgtp2, mx, zrow, zden)


def _epi_body(hp_ref, den_ref, out_ref):
    hsum = hp_ref[0] + hp_ref[1]
    den = den_ref[0] + den_ref[1] + 1e-10
    h = hsum / den
    out_ref[...] = jnp.where(h > 0.0, h, jnp.exp(h) - 1.0)


def _epilogue(hp, den, n_rows, blk):
    grid = (n_rows // blk,)
    return pl.pallas_call(
        _epi_body,
        grid=grid,
        in_specs=[
            pl.BlockSpec((2, blk, D), lambda i: (0, i, 0)),
            pl.BlockSpec((2, blk, 1), lambda i: (0, i, 0)),
        ],
        out_specs=pl.BlockSpec((blk, D), lambda i: (i, 0)),
        out_shape=jax.ShapeDtypeStruct((n_rows, D), jnp.float32),
    )(hp, den)


def kernel(node_features, edge_index, W, a):
    x = node_features
    wt = W.T
    a2 = jnp.reshape(a, (2, D)).T

    h, sca = _project(x, wt, a2)
    s1 = sca[:, 0]
    s2 = sca[:, 1]

    src = jnp.reshape(edge_index[0], (NW, EPW))
    tgt = jnp.reshape(edge_index[1], (NW, EPW))
    srcp = jnp.pad(src, ((0, 0), (0, EPW_PAD - EPW)))
    tgtp = jnp.pad(tgt, ((0, 0), (0, EPW_PAD - EPW)), constant_values=N)

    e, mx = _logits(s1, s2, srcp, tgtp)

    srcp2 = jnp.reshape(srcp, (NW, GROUPS, GW))
    tgtp2 = jnp.reshape(tgtp, (NW, GROUPS, GW))
    zrow = jnp.zeros((NROW_PAD, D), jnp.float32)
    zden = jnp.zeros((NROW_PAD,), jnp.float32)
    hp, den = _aggregate(h, e, srcp2, tgtp2, mx, zrow, zden)

    out = _epilogue(hp, den[:, :, None], NROW_PAD, 1024)
    return out[:N]


# speedup vs baseline: 18.7641x; 1.0949x over previous
import functools
import jax
import jax.numpy as jnp
from jax import lax
from jax.experimental import pallas as pl
from jax.experimental.pallas import tpu as pltpu
from jax.experimental.pallas import tpu_sc as plsc

N = 10000
E = 320000
D = 128
ALPHA = 0.2

NC = 2
NS = 16
NW = NC * NS

EPW = E // NW
GW = 128
GROUPS = 79
EPW_PAD = GROUPS * GW
NROW_PAD = 10240
RPT = NROW_PAD // NS

N_BLK = 1000

_NEG = -1e30


def _proj_body(x_ref, wt_ref, a_ref, h_ref, s_ref):
    h = jnp.dot(x_ref[...], wt_ref[...], preferred_element_type=jnp.float32)
    h_ref[...] = h
    s_ref[...] = jnp.dot(h, a_ref[...], preferred_element_type=jnp.float32)


def _project(x, wt, a2):
    grid = (N // N_BLK,)
    return pl.pallas_call(
        _proj_body,
        grid=grid,
        in_specs=[
            pl.BlockSpec((N_BLK, D), lambda i: (i, 0)),
            pl.BlockSpec((D, D), lambda i: (0, 0)),
            pl.BlockSpec((D, 2), lambda i: (0, 0)),
        ],
        out_specs=[
            pl.BlockSpec((N_BLK, D), lambda i: (i, 0)),
            pl.BlockSpec((N_BLK, 2), lambda i: (i, 0)),
        ],
        out_shape=[
            jax.ShapeDtypeStruct((N, D), jnp.float32),
            jax.ShapeDtypeStruct((N, 2), jnp.float32),
        ],
    )(x, wt, a2)


def _logits_body(s1_hbm, s2_hbm, src_hbm, tgt_hbm, e_hbm, mx_hbm,
                 s1_v, s2_v, src_v, tgt_v, e_v, mx_v):
    c = lax.axis_index("c")
    s = lax.axis_index("s")
    wid = c * NS + s

    pltpu.sync_copy(s1_hbm, s1_v)
    pltpu.sync_copy(s2_hbm, s2_v)
    pltpu.sync_copy(src_hbm.at[wid], src_v)
    pltpu.sync_copy(tgt_hbm.at[wid], tgt_v)

    def one(off, macc):
        sv = src_v[pl.ds(off, 16)]
        tv = tgt_v[pl.ds(off, 16)]
        v = plsc.load_gather(s1_v, [sv]) + plsc.load_gather(s2_v, [tv])
        e = jnp.maximum(v, ALPHA * v)
        e_v[pl.ds(off, 16)] = e
        return jnp.maximum(macc, e)

    def body(j, macc):
        base = j * 128
        for k in range(8):
            macc = one(base + 16 * k, macc)
        return macc

    macc = jnp.full((16,), _NEG, jnp.float32)
    macc = lax.fori_loop(0, EPW // 128, body, macc)
    macc = one(EPW - 16, macc)
    pad = jnp.full((16,), _NEG, jnp.float32)
    for k in range((EPW_PAD - EPW) // 16):
        e_v[pl.ds(EPW + 16 * k, 16)] = pad

    mx_v[...] = macc
    pltpu.sync_copy(e_v, e_hbm.at[wid])
    pltpu.sync_copy(mx_v, mx_hbm.at[pl.ds(wid * 16, 16)])


def _logits(s1, s2, srcp, tgtp):
    mesh = plsc.VectorSubcoreMesh(core_axis_name="c", subcore_axis_name="s",
                                  num_cores=NC, num_subcores=NS)
    f = pl.kernel(
        _logits_body,
        out_type=[
            jax.ShapeDtypeStruct((NW, EPW_PAD), jnp.float32),
            jax.ShapeDtypeStruct((NW * 16,), jnp.float32),
        ],
        mesh=mesh,
        scratch_types=[
            pltpu.VMEM((N,), jnp.float32),
            pltpu.VMEM((N,), jnp.float32),
            pltpu.VMEM((EPW_PAD,), jnp.int32),
            pltpu.VMEM((EPW_PAD,), jnp.int32),
            pltpu.VMEM((EPW_PAD,), jnp.float32),
            pltpu.VMEM((16,), jnp.float32),
        ],
        compiler_params=pltpu.CompilerParams(needs_layout_passes=False),
    )
    return f(s1, s2, srcp, tgtp)


def _agg_body(h_hbm, e_hbm, src_hbm, tgt_hbm, mx_hbm,
              hp_hbm, den_hbm,
              mx_v, rows_a, rows_b, srcg_a, srcg_b, tgtg_a, tgtg_b,
              eg_a, eg_b, pg_a, pg_b, stg_a, stg_b, spg_a, spg_b,
              hp_sh, den_sh,
              gsa, gsb, psa, psb, ssa, ssb, dsem):
    c = lax.axis_index("c")
    s = lax.axis_index("s")
    wid = c * NS + s

    pltpu.sync_copy(mx_hbm, mx_v)

    def mx_body(i, macc):
        return jnp.maximum(macc, mx_v[pl.ds(i * 16, 16)])

    macc = lax.fori_loop(0, NW, mx_body, jnp.full((16,), _NEG, jnp.float32))
    m = jnp.max(macc)

    zv = jnp.zeros((16,), jnp.float32)

    def zrow_body(r, _):
        for k in range(8):
            rows_a[r, pl.ds(16 * k, 16)] = zv
        return 0

    lax.fori_loop(0, GW, zrow_body, 0)
    for k in range(GW // 16):
        pg_a[pl.ds(16 * k, 16)] = zv
    for q in range(RPT // GW):
        pltpu.sync_copy(rows_a, hp_sh.at[pl.ds(s * RPT + q * GW, GW)])
        pltpu.sync_copy(pg_a, den_sh.at[pl.ds(s * RPT + q * GW, GW)])
    plsc.subcore_barrier()

    bufs = ((rows_a, srcg_a, tgtg_a, eg_a, pg_a, gsa, psa, stg_a, spg_a, ssa),
            (rows_b, srcg_b, tgtg_b, eg_b, pg_b, gsb, psb, stg_b, spg_b, ssb))

    def pf_issue(g, bb, sem):
        pltpu.async_copy(src_hbm.at[wid, g], bb[1], sem)
        pltpu.async_copy(tgt_hbm.at[wid, g], bb[2], sem)
        pltpu.async_copy(e_hbm.at[wid, pl.ds(g * GW, GW)], bb[3], sem)

    def pf_wait(g, bb, sem):
        pltpu.make_async_copy(src_hbm.at[wid, g], bb[1], sem).wait()
        pltpu.make_async_copy(tgt_hbm.at[wid, g], bb[2], sem).wait()
        pltpu.make_async_copy(e_hbm.at[wid, pl.ds(g * GW, GW)], bb[3],
                              sem).wait()

    def p_transform(bb):
        for q in range(GW // 16):
            sl = pl.ds(16 * q, 16)
            bb[4][sl] = jnp.exp(bb[3][sl] - m)

    def g_issue(g, bb):
        pltpu.async_copy(h_hbm.at[bb[1]], bb[0], bb[5])

    def g_wait(g, bb):
        pltpu.make_async_copy(h_hbm.at[bb[1]], bb[0], bb[5]).wait()

    def d_issue(bb):
        pltpu.async_copy(bb[8], den_sh.at[bb[7]], dsem, add=True)

    def d_wait(bb):
        pltpu.make_async_copy(bb[8], den_sh.at[bb[7]], dsem).wait()

    def shadow_copy(bb):
        for k in range(GW // 16):
            sl = pl.ds(16 * k, 16)
            bb[7][sl] = bb[2][sl]
            bb[8][sl] = bb[4][sl]

    def scale(bb):
        rows, pg = bb[0], bb[4]

        def row16(mm, _):
            base = mm * 16
            pvec = pg[pl.ds(base, 16)]
            for j in range(16):
                r = base + j
                sc = jnp.full((16,), pvec[j])
                for k in range(8):
                    sl = pl.ds(16 * k, 16)
                    rows[r, sl] = rows[r, sl] * sc
            return 0

        lax.fori_loop(0, GW // 16, row16, 0)

    def s_issue(bb):
        pltpu.async_copy(bb[0], hp_sh.at[bb[7]], bb[9], add=True)

    def s_wait(bb):
        pltpu.make_async_copy(bb[0], hp_sh.at[bb[7]], bb[9]).wait()

    pf_issue(0, bufs[0], bufs[0][6])
    pf_wait(0, bufs[0], bufs[0][6])
    p_transform(bufs[0])
    g_issue(0, bufs[0])
    pf_issue(1, bufs[1], bufs[1][6])

    def phase(g, bx, by):
        g_wait(g, bx)

        @pl.when(g + 1 < GROUPS)
        def _():
            pf_wait(g + 1, by, by[6])
            p_transform(by)

            @pl.when(g >= 1)
            def _():
                s_wait(by)
                d_wait(by)
            g_issue(g + 1, by)

        shadow_copy(bx)
        d_issue(bx)
        scale(bx)
        s_issue(bx)

        @pl.when(g + 2 < GROUPS)
        def _():
            pf_issue(g + 2, bx, bx[6])

    def pair(gg, _):
        g0 = 2 * gg
        phase(g0, bufs[0], bufs[1])

        @pl.when(g0 + 1 < GROUPS)
        def _():
            phase(g0 + 1, bufs[1], bufs[0])

        return 0

    lax.fori_loop(0, (GROUPS + 1) // 2, pair, 0)
    s_wait(bufs[0])
    s_wait(bufs[1])
    d_wait(bufs[0])
    d_wait(bufs[1])
    plsc.subcore_barrier()

    rslc = pl.ds(s * RPT, RPT)
    pltpu.sync_copy(hp_sh.at[rslc], hp_hbm.at[c, rslc])
    pltpu.sync_copy(den_sh.at[rslc], den_hbm.at[c, rslc])


def _aggregate(h, e, srcp2, tgtp2, mx):
    mesh = plsc.VectorSubcoreMesh(core_axis_name="c", subcore_axis_name="s",
                                  num_cores=NC, num_subcores=NS)
    f = pl.kernel(
        _agg_body,
        out_type=[
            jax.ShapeDtypeStruct((NC, NROW_PAD, D), jnp.float32),
            jax.ShapeDtypeStruct((NC, NROW_PAD), jnp.float32),
        ],
        mesh=mesh,
        scratch_types=[
            pltpu.VMEM((NW * 16,), jnp.float32),
            pltpu.VMEM((GW, D), jnp.float32),
            pltpu.VMEM((GW, D), jnp.float32),
            pltpu.VMEM((GW,), jnp.int32),
            pltpu.VMEM((GW,), jnp.int32),
            pltpu.VMEM((GW,), jnp.int32),
            pltpu.VMEM((GW,), jnp.int32),
            pltpu.VMEM((GW,), jnp.float32),
            pltpu.VMEM((GW,), jnp.float32),
            pltpu.VMEM((GW,), jnp.float32),
            pltpu.VMEM((GW,), jnp.float32),
            pltpu.VMEM((GW,), jnp.int32),
            pltpu.VMEM((GW,), jnp.int32),
            pltpu.VMEM((GW,), jnp.float32),
            pltpu.VMEM((GW,), jnp.float32),
            pltpu.VMEM_SHARED((NROW_PAD, D), jnp.float32),
            pltpu.VMEM_SHARED((NROW_PAD,), jnp.float32),
            pltpu.SemaphoreType.DMA,
            pltpu.SemaphoreType.DMA,
            pltpu.SemaphoreType.DMA,
            pltpu.SemaphoreType.DMA,
            pltpu.SemaphoreType.DMA,
            pltpu.SemaphoreType.DMA,
            pltpu.SemaphoreType.DMA,
        ],
        compiler_params=pltpu.CompilerParams(needs_layout_passes=False),
    )
    return f(h, e, srcp2, tgtp2, mx)


def _epi_body(hp_ref, den_ref, out_ref):
    hsum = hp_ref[0] + hp_ref[1]
    den = den_ref[0] + den_ref[1] + 1e-10
    h = hsum / den
    out_ref[...] = jnp.where(h > 0.0, h, jnp.exp(h) - 1.0)


def _epilogue(hp, den, n_rows, blk):
    grid = (n_rows // blk,)
    return pl.pallas_call(
        _epi_body,
        grid=grid,
        in_specs=[
            pl.BlockSpec((2, blk, D), lambda i: (0, i, 0)),
            pl.BlockSpec((2, blk, 1), lambda i: (0, i, 0)),
        ],
        out_specs=pl.BlockSpec((blk, D), lambda i: (i, 0)),
        out_shape=jax.ShapeDtypeStruct((n_rows, D), jnp.float32),
    )(hp, den)


def kernel(node_features, edge_index, W, a):
    x = node_features
    wt = W.T
    a2 = jnp.reshape(a, (2, D)).T

    h, sca = _project(x, wt, a2)
    s1 = sca[:, 0]
    s2 = sca[:, 1]

    src = jnp.reshape(edge_index[0], (NW, EPW))
    tgt = jnp.reshape(edge_index[1], (NW, EPW))
    srcp = jnp.pad(src, ((0, 0), (0, EPW_PAD - EPW)))
    tgtp = jnp.pad(tgt, ((0, 0), (0, EPW_PAD - EPW)), constant_values=N)

    e, mx = _logits(s1, s2, srcp, tgtp)

    srcp2 = jnp.reshape(srcp, (NW, GROUPS, GW))
    tgtp2 = jnp.reshape(tgtp, (NW, GROUPS, GW))
    hp, den = _aggregate(h, e, srcp2, tgtp2, mx)

    return _epilogue(hp, den[:, :, None], N, N_BLK)


# speedup vs baseline: 19.4765x; 1.0380x over previous
import functools
import jax
import jax.numpy as jnp
from jax import lax
from jax.experimental import pallas as pl
from jax.experimental.pallas import tpu as pltpu
from jax.experimental.pallas import tpu_sc as plsc

N = 10000
E = 320000
D = 128
ALPHA = 0.2

NC = 2
NS = 16
NW = NC * NS

EPW = E // NW
GW = 64
GROUPS = 158
EPW_PAD = GROUPS * GW
NBUF = 4
NROW_PAD = 10240
RPT = NROW_PAD // NS

N_BLK = 1000

_NEG = -1e30


def _proj_body(x_ref, wt_ref, a_ref, h_ref, s_ref):
    h = jnp.dot(x_ref[...], wt_ref[...], preferred_element_type=jnp.float32)
    h_ref[...] = h
    s_ref[...] = jnp.dot(h, a_ref[...], preferred_element_type=jnp.float32)


def _project(x, wt, a2):
    grid = (N // N_BLK,)
    return pl.pallas_call(
        _proj_body,
        grid=grid,
        in_specs=[
            pl.BlockSpec((N_BLK, D), lambda i: (i, 0)),
            pl.BlockSpec((D, D), lambda i: (0, 0)),
            pl.BlockSpec((D, 2), lambda i: (0, 0)),
        ],
        out_specs=[
            pl.BlockSpec((N_BLK, D), lambda i: (i, 0)),
            pl.BlockSpec((N_BLK, 2), lambda i: (i, 0)),
        ],
        out_shape=[
            jax.ShapeDtypeStruct((N, D), jnp.float32),
            jax.ShapeDtypeStruct((N, 2), jnp.float32),
        ],
    )(x, wt, a2)


def _logits_body(s1_hbm, s2_hbm, src_hbm, tgt_hbm, e_hbm, mx_hbm,
                 s1_v, s2_v, src_v, tgt_v, e_v, mx_v):
    c = lax.axis_index("c")
    s = lax.axis_index("s")
    wid = c * NS + s

    pltpu.sync_copy(s1_hbm, s1_v)
    pltpu.sync_copy(s2_hbm, s2_v)
    pltpu.sync_copy(src_hbm.at[wid], src_v)
    pltpu.sync_copy(tgt_hbm.at[wid], tgt_v)

    def one(off, macc):
        sv = src_v[pl.ds(off, 16)]
        tv = tgt_v[pl.ds(off, 16)]
        v = plsc.load_gather(s1_v, [sv]) + plsc.load_gather(s2_v, [tv])
        e = jnp.maximum(v, ALPHA * v)
        e_v[pl.ds(off, 16)] = e
        return jnp.maximum(macc, e)

    def body(j, macc):
        base = j * 128
        for k in range(8):
            macc = one(base + 16 * k, macc)
        return macc

    macc = jnp.full((16,), _NEG, jnp.float32)
    macc = lax.fori_loop(0, EPW // 128, body, macc)
    macc = one(EPW - 16, macc)
    pad = jnp.full((16,), _NEG, jnp.float32)
    for k in range((EPW_PAD - EPW) // 16):
        e_v[pl.ds(EPW + 16 * k, 16)] = pad

    mx_v[...] = macc
    pltpu.sync_copy(e_v, e_hbm.at[wid])
    pltpu.sync_copy(mx_v, mx_hbm.at[pl.ds(wid * 16, 16)])


def _logits(s1, s2, srcp, tgtp):
    mesh = plsc.VectorSubcoreMesh(core_axis_name="c", subcore_axis_name="s",
                                  num_cores=NC, num_subcores=NS)
    f = pl.kernel(
        _logits_body,
        out_type=[
            jax.ShapeDtypeStruct((NW, EPW_PAD), jnp.float32),
            jax.ShapeDtypeStruct((NW * 16,), jnp.float32),
        ],
        mesh=mesh,
        scratch_types=[
            pltpu.VMEM((N,), jnp.float32),
            pltpu.VMEM((N,), jnp.float32),
            pltpu.VMEM((EPW_PAD,), jnp.int32),
            pltpu.VMEM((EPW_PAD,), jnp.int32),
            pltpu.VMEM((EPW_PAD,), jnp.float32),
            pltpu.VMEM((16,), jnp.float32),
        ],
        compiler_params=pltpu.CompilerParams(needs_layout_passes=False),
    )
    return f(s1, s2, srcp, tgtp)


def _agg_body(h_hbm, e_hbm, src_hbm, tgt_hbm, mx_hbm,
              hp_hbm, den_hbm, mx_v, *scr):
    bufs = tuple(scr[7 * i:7 * i + 7] for i in range(NBUF))
    hp_sh = scr[7 * NBUF]
    den_sh = scr[7 * NBUF + 1]
    gsem = scr[7 * NBUF + 2:7 * NBUF + 2 + NBUF]
    psem = scr[7 * NBUF + 2 + NBUF:7 * NBUF + 2 + 2 * NBUF]
    ssem = scr[7 * NBUF + 2 + 2 * NBUF:7 * NBUF + 2 + 3 * NBUF]
    dsem = scr[7 * NBUF + 2 + 3 * NBUF]

    c = lax.axis_index("c")
    s = lax.axis_index("s")
    wid = c * NS + s

    pltpu.sync_copy(mx_hbm, mx_v)

    def mx_body(i, macc):
        return jnp.maximum(macc, mx_v[pl.ds(i * 16, 16)])

    macc = lax.fori_loop(0, NW, mx_body, jnp.full((16,), _NEG, jnp.float32))
    m = jnp.max(macc)

    zv = jnp.zeros((16,), jnp.float32)
    rows0, pg0 = bufs[0][0], bufs[0][4]

    def zrow_body(r, _):
        for k in range(D // 16):
            rows0[r, pl.ds(16 * k, 16)] = zv
        return 0

    lax.fori_loop(0, GW, zrow_body, 0)
    for k in range(GW // 16):
        pg0[pl.ds(16 * k, 16)] = zv
    for q in range(RPT // GW):
        pltpu.sync_copy(rows0, hp_sh.at[pl.ds(s * RPT + q * GW, GW)])
        pltpu.sync_copy(pg0, den_sh.at[pl.ds(s * RPT + q * GW, GW)])
    plsc.subcore_barrier()

    def pf_issue(g, j):
        bb = bufs[j]
        pltpu.async_copy(src_hbm.at[wid, g], bb[1], psem[j])
        pltpu.async_copy(tgt_hbm.at[wid, g], bb[2], psem[j])
        pltpu.async_copy(e_hbm.at[wid, pl.ds(g * GW, GW)], bb[3], psem[j])

    def pf_wait(g, j):
        bb = bufs[j]
        pltpu.make_async_copy(src_hbm.at[wid, g], bb[1], psem[j]).wait()
        pltpu.make_async_copy(tgt_hbm.at[wid, g], bb[2], psem[j]).wait()
        pltpu.make_async_copy(e_hbm.at[wid, pl.ds(g * GW, GW)], bb[3],
                              psem[j]).wait()

    def p_transform(j):
        bb = bufs[j]
        for q in range(GW // 16):
            sl = pl.ds(16 * q, 16)
            bb[4][sl] = jnp.exp(bb[3][sl] - m)

    def g_issue(g, j):
        pltpu.async_copy(h_hbm.at[bufs[j][1]], bufs[j][0], gsem[j])

    def g_wait(g, j):
        pltpu.make_async_copy(h_hbm.at[bufs[j][1]], bufs[j][0],
                              gsem[j]).wait()

    def d_issue(j):
        pltpu.async_copy(bufs[j][6], den_sh.at[bufs[j][5]], dsem, add=True)

    def d_wait(j):
        pltpu.make_async_copy(bufs[j][6], den_sh.at[bufs[j][5]],
                              dsem).wait()

    def shadow_copy(j):
        bb = bufs[j]
        for k in range(GW // 16):
            sl = pl.ds(16 * k, 16)
            bb[5][sl] = bb[2][sl]
            bb[6][sl] = bb[4][sl]

    def scale(j):
        rows, pg = bufs[j][0], bufs[j][4]

        def row16(mm, _):
            base = mm * 16
            pvec = pg[pl.ds(base, 16)]
            for jj in range(16):
                r = base + jj
                sc = jnp.full((16,), pvec[jj])
                for k in range(D // 16):
                    sl = pl.ds(16 * k, 16)
                    rows[r, sl] = rows[r, sl] * sc
            return 0

        lax.fori_loop(0, GW // 16, row16, 0)

    def s_issue(j):
        pltpu.async_copy(bufs[j][0], hp_sh.at[bufs[j][5]], ssem[j],
                         add=True)

    def s_wait(j):
        pltpu.make_async_copy(bufs[j][0], hp_sh.at[bufs[j][5]],
                              ssem[j]).wait()

    for i in range(NBUF):
        pf_issue(i, i)
    for i in range(NBUF - 1):
        pf_wait(i, i)
        g_issue(i, i)

    def phase(g, j):
        jp = (j - 1) % NBUF
        g_wait(g, j)
        p_transform(j)

        @pl.when(g >= NBUF)
        def _():
            d_wait(j)

        shadow_copy(j)
        d_issue(j)
        scale(j)
        s_issue(j)

        @pl.when(g + NBUF - 1 < GROUPS)
        def _():
            @pl.when(g >= 1)
            def _():
                s_wait(jp)
            pf_wait(g + NBUF - 1, jp)
            g_issue(g + NBUF - 1, jp)

        @pl.when(g + NBUF < GROUPS)
        def _():
            pf_issue(g + NBUF, j)

    def stride(t, _):
        base = t * NBUF
        for j in range(NBUF):
            g = base + j

            @pl.when(g < GROUPS)
            def _():
                phase(g, j)

        return 0

    lax.fori_loop(0, (GROUPS + NBUF - 1) // NBUF, stride, 0)
    for j in range(NBUF):
        s_wait(j)
        d_wait(j)
    plsc.subcore_barrier()

    rslc = pl.ds(s * RPT, RPT)
    pltpu.sync_copy(hp_sh.at[rslc], hp_hbm.at[c, rslc])
    pltpu.sync_copy(den_sh.at[rslc], den_hbm.at[c, rslc])


def _aggregate(h, e, srcp2, tgtp2, mx):
    mesh = plsc.VectorSubcoreMesh(core_axis_name="c", subcore_axis_name="s",
                                  num_cores=NC, num_subcores=NS)
    per_buf = [
        pltpu.VMEM((GW, D), jnp.float32),
        pltpu.VMEM((GW,), jnp.int32),
        pltpu.VMEM((GW,), jnp.int32),
        pltpu.VMEM((GW,), jnp.float32),
        pltpu.VMEM((GW,), jnp.float32),
        pltpu.VMEM((GW,), jnp.int32),
        pltpu.VMEM((GW,), jnp.float32),
    ]
    scratch = [pltpu.VMEM((NW * 16,), jnp.float32)]
    for _ in range(NBUF):
        scratch.extend(per_buf)
    scratch.append(pltpu.VMEM_SHARED((NROW_PAD, D), jnp.float32))
    scratch.append(pltpu.VMEM_SHARED((NROW_PAD,), jnp.float32))
    scratch.extend([pltpu.SemaphoreType.DMA] * (3 * NBUF + 1))
    f = pl.kernel(
        _agg_body,
        out_type=[
            jax.ShapeDtypeStruct((NC, NROW_PAD, D), jnp.float32),
            jax.ShapeDtypeStruct((NC, NROW_PAD), jnp.float32),
        ],
        mesh=mesh,
        scratch_types=scratch,
        compiler_params=pltpu.CompilerParams(needs_layout_passes=False),
    )
    return f(h, e, srcp2, tgtp2, mx)


def _epi_body(hp_ref, den_ref, out_ref):
    hsum = hp_ref[0] + hp_ref[1]
    den = den_ref[0] + den_ref[1] + 1e-10
    h = hsum / den
    out_ref[...] = jnp.where(h > 0.0, h, jnp.exp(h) - 1.0)


def _epilogue(hp, den, n_rows, blk):
    grid = (n_rows // blk,)
    return pl.pallas_call(
        _epi_body,
        grid=grid,
        in_specs=[
            pl.BlockSpec((2, blk, D), lambda i: (0, i, 0)),
            pl.BlockSpec((2, blk, 1), lambda i: (0, i, 0)),
        ],
        out_specs=pl.BlockSpec((blk, D), lambda i: (i, 0)),
        out_shape=jax.ShapeDtypeStruct((n_rows, D), jnp.float32),
    )(hp, den)


def kernel(node_features, edge_index, W, a):
    x = node_features
    wt = W.T
    a2 = jnp.reshape(a, (2, D)).T

    h, sca = _project(x, wt, a2)
    s1 = sca[:, 0]
    s2 = sca[:, 1]

    src = jnp.reshape(edge_index[0], (NW, EPW))
    tgt = jnp.reshape(edge_index[1], (NW, EPW))
    srcp = jnp.pad(src, ((0, 0), (0, EPW_PAD - EPW)))
    tgtp = jnp.pad(tgt, ((0, 0), (0, EPW_PAD - EPW)), constant_values=N)

    e, mx = _logits(s1, s2, srcp, tgtp)

    srcp2 = jnp.reshape(srcp, (NW, GROUPS, GW))
    tgtp2 = jnp.reshape(tgtp, (NW, GROUPS, GW))
    hp, den = _aggregate(h, e, srcp2, tgtp2, mx)

    return _epilogue(hp, den[:, :, None], N, N_BLK)
